# baseline (speedup 1.0000x reference)
import jax
import jax.numpy as jnp
from jax import lax
from jax.experimental import pallas as pl
from jax.experimental.pallas import tpu as pltpu
from jax.experimental.pallas import tpu_sc as plsc

B = 16384
D = 64
VOCAB = 1000000
NC = 2
NS = 16
NW = NC * NS
BPW = B // NW
CH = 128
NCH = BPW // CH
G = 16
NG = BPW // G

_mesh = plsc.VectorSubcoreMesh(core_axis_name="c", subcore_axis_name="s",
                               num_cores=NC, num_subcores=NS)


def _sc_body(x_hbm, ids_hbm, V_hbm, out_hbm, idx_v, rows_v, x_v, S, out_v, sem):
    wid = lax.axis_index("s") * NC + lax.axis_index("c")
    base = wid * BPW

    pltpu.sync_copy(ids_hbm.at[wid], idx_v)
    gathers = [
        pltpu.async_copy(V_hbm.at[idx_v.at[c]], rows_v.at[pl.ds(c * CH, CH)], sem)
        for c in range(NCH)
    ]
    pltpu.sync_copy(x_hbm.at[pl.ds(base, BPW)], x_v)
    for g in gathers:
        g.wait()

    lanes = lax.iota(jnp.int32, 16)

    def group(g, carry):
        row0 = g * G
        for r in range(G):
            row = row0 + r
            p = rows_v[row, pl.ds(0, 16)] * x_v[row, pl.ds(0, 16)]
            for j in range(1, D // 16):
                p = p + rows_v[row, pl.ds(j * 16, 16)] * x_v[row, pl.ds(j * 16, 16)]
            S[pl.ds(r * 17, 16)] = p
        cols = lanes * 17
        acc = plsc.load_gather(S, [cols])
        for c in range(1, 16):
            acc = acc + plsc.load_gather(S, [cols + c])
        out_v[pl.ds(row0, G)] = 1.0 / (1.0 + jnp.exp(-acc))
        return carry

    lax.fori_loop(0, NG, group, 0)
    pltpu.sync_copy(out_v, out_hbm.at[pl.ds(base, BPW)])


_sc_kernel = pl.kernel(
    _sc_body,
    out_type=jax.ShapeDtypeStruct((B,), jnp.float32),
    mesh=_mesh,
    compiler_params=pltpu.CompilerParams(needs_layout_passes=False,
                                         use_tc_tiling_on_sc=False),
    scratch_types=[
        pltpu.VMEM((NCH, CH), jnp.int32),
        pltpu.VMEM((BPW, D), jnp.float32),
        pltpu.VMEM((BPW, D), jnp.float32),
        pltpu.VMEM((G * 17,), jnp.float32),
        pltpu.VMEM((BPW,), jnp.float32),
        pltpu.SemaphoreType.DMA,
    ],
)


@jax.jit
def kernel(x, emoji_ids, V):
    ids = emoji_ids.astype(jnp.int32).reshape(NW, NCH, CH)
    return _sc_kernel(x, ids, V)


# speedup vs baseline: 1.6720x; 1.6720x over previous
import jax
import jax.numpy as jnp
from jax import lax
from jax.experimental import pallas as pl
from jax.experimental.pallas import tpu as pltpu
from jax.experimental.pallas import tpu_sc as plsc

B = 16384
D = 64
VOCAB = 1000000
NC = 2
NS = 16
NW = NC * NS
BPW = B // NW
HB = BPW // 2
UNROLL = 8
G = 16
NG = BPW // G

_mesh = plsc.VectorSubcoreMesh(core_axis_name="c", subcore_axis_name="s",
                               num_cores=NC, num_subcores=NS)


def _sc_body(x_hbm, ids_hbm, V_hbm, out_hbm,
             ids_v, rows_v, x_v, S, out_v, sem, xsem):
    wid = lax.axis_index("s") * NC + lax.axis_index("c")
    base = wid * BPW

    pltpu.sync_copy(ids_hbm.at[wid], ids_v)
    xcp = pltpu.async_copy(x_hbm.at[pl.ds(base, HB)], x_v, xsem)

    lanes16 = lax.iota(jnp.int32, 16)
    masks = [lanes16 == r for r in range(16)]
    zeros16 = jnp.zeros((16,), jnp.int32)

    def enq(i, carry):
        idvec = ids_v[pl.ds(i * 16, 16)]
        for u in range(16):
            sid = jnp.sum(jnp.where(masks[u], idvec, zeros16))
            pltpu.async_copy(V_hbm.at[sid], rows_v.at[i * 16 + u], sem)
        return carry

    lax.fori_loop(0, BPW // 16, enq, 0)
    pltpu.make_async_copy(V_hbm.at[pl.ds(0, BPW)], rows_v, sem).wait()

    lanes = lax.iota(jnp.int32, 16)
    cols = lanes * 17

    def group(g, carry):
        row0 = g * G
        xrow0 = row0 - (g // (HB // G)) * HB
        for r in range(G):
            row = row0 + r
            xrow = xrow0 + r
            p = rows_v[row, pl.ds(0, 16)] * x_v[xrow, pl.ds(0, 16)]
            for j in range(1, D // 16):
                p = p + rows_v[row, pl.ds(j * 16, 16)] * x_v[xrow, pl.ds(j * 16, 16)]
            S[pl.ds(r * 17, 16)] = p
        acc = plsc.load_gather(S, [cols])
        for c in range(1, 16):
            acc = acc + plsc.load_gather(S, [cols + c])
        out_v[pl.ds(row0, G)] = 1.0 / (1.0 + jnp.exp(-acc))
        return carry

    xcp.wait()
    lax.fori_loop(0, NG // 2, group, 0)
    pltpu.sync_copy(x_hbm.at[pl.ds(base + HB, HB)], x_v)
    lax.fori_loop(NG // 2, NG, group, 0)
    pltpu.sync_copy(out_v, out_hbm.at[pl.ds(base, BPW)])


_sc_kernel = pl.kernel(
    _sc_body,
    out_type=jax.ShapeDtypeStruct((B,), jnp.float32),
    mesh=_mesh,
    compiler_params=pltpu.CompilerParams(needs_layout_passes=False),
    scratch_types=[
        pltpu.VMEM((BPW,), jnp.int32),
        pltpu.VMEM((BPW, D), jnp.float32),
        pltpu.VMEM((HB, D), jnp.float32),
        pltpu.VMEM((G * 17,), jnp.float32),
        pltpu.VMEM((BPW,), jnp.float32),
        pltpu.SemaphoreType.DMA,
        pltpu.SemaphoreType.DMA,
    ],
)


@jax.jit
def kernel(x, emoji_ids, V):
    ids = emoji_ids.astype(jnp.int32).reshape(NW, BPW)
    return _sc_kernel(x, ids, V)
